# baseline (speedup 1.0000x reference)
import functools

import jax
import jax.numpy as jnp
from jax import lax
from jax.experimental import pallas as pl
from jax.experimental.pallas import tpu as pltpu
from jax.experimental.pallas import tpu_sc as plsc

VOCAB = 1000000
D = 64
DP = 128
B = 4096
L = 200
N = B * L

NC = 2
NS = 16
NW = NC * NS
PER_W = N // NW
NBUF = 2
CHUNK = 128
GRP = CHUNK // 16
N_CHUNKS = PER_W // CHUNK
N_ITERS = N_CHUNKS // NBUF


def _emb_body(idx_hbm, lbl_hbm, emb_hbm, seg_hbm, out_hbm,
              idx_v, lbl_v, tok_v, seg_v, sem_g, sem_s, sem_w):
    wid = lax.axis_index("s") * NC + lax.axis_index("c")
    base = wid * PER_W

    with jax.named_scope("stg_idx"):
        pltpu.sync_copy(idx_hbm.at[pl.ds(base, PER_W)], idx_v)
        pltpu.sync_copy(lbl_hbm.at[pl.ds(base, PER_W)], lbl_v)

    def fire(g, b):
        off = g * CHUNK

        def fire_one(p, c):
            sl = pl.ds(off + p * 16, 16)
            dl = pl.ds(p * 16, 16)
            pltpu.async_copy(seg_hbm.at[lbl_v[sl]], seg_v.at[b, dl],
                             sem_s.at[b])
            pltpu.async_copy(emb_hbm.at[idx_v[sl]], tok_v.at[b, dl],
                             sem_g.at[b])
            return c

        with jax.named_scope("fire"):
            lax.fori_loop(0, GRP, fire_one, 0)

    def finish(g, b):
        with jax.named_scope("drain"):
            pltpu.make_async_copy(seg_hbm.at[lbl_v.at[pl.ds(0, CHUNK)]],
                                  seg_v.at[b], sem_s.at[b]).wait()
            pltpu.make_async_copy(emb_hbm.at[idx_v.at[pl.ds(0, CHUNK)]],
                                  tok_v.at[b], sem_g.at[b]).wait()

        def add_step(p, c):
            for k in range(D // 16):
                sl = pl.ds(k * 16, 16)
                tok_v[b, p, sl] = tok_v[b, p, sl] + seg_v[b, p, sl]
            return c

        with jax.named_scope("addloop"):
            lax.fori_loop(0, CHUNK, add_step, 0, unroll=2)
        start = base + g * CHUNK
        with jax.named_scope("wb"):
            pltpu.async_copy(tok_v.at[b], out_hbm.at[pl.ds(start, CHUNK)],
                             sem_w.at[b]).wait()

    fire(0, 0)

    def step(i, carry):
        g0 = i * NBUF
        fire(g0 + 1, 1)
        finish(g0, 0)

        @pl.when(i + 1 < N_ITERS)
        def _():
            fire(g0 + 2, 0)

        finish(g0 + 1, 1)
        return carry

    lax.fori_loop(0, N_ITERS, step, 0)


@jax.jit
def _emb_lookup(idx, lbl, emb_table, seg_table):
    embp = jnp.pad(emb_table, ((0, 0), (0, DP - D)))
    segp = jnp.pad(seg_table, ((0, 0), (0, DP - D)))
    mesh = plsc.VectorSubcoreMesh(core_axis_name="c", subcore_axis_name="s")
    f = pl.kernel(
        _emb_body,
        out_type=jax.ShapeDtypeStruct((N, DP), jnp.float32),
        mesh=mesh,
        scratch_types=[
            pltpu.VMEM((PER_W,), jnp.int32),
            pltpu.VMEM((PER_W,), jnp.int32),
            pltpu.VMEM((NBUF, CHUNK, DP), jnp.float32),
            pltpu.VMEM((NBUF, CHUNK, DP), jnp.float32),
            pltpu.SemaphoreType.DMA((NBUF,)),
            pltpu.SemaphoreType.DMA((NBUF,)),
            pltpu.SemaphoreType.DMA((NBUF,)),
        ],
    )
    return f(idx, lbl, embp, segp)


def kernel(x, segment_label, emb_table, seg_table):
    idx = x.reshape(-1).astype(jnp.int32)
    lbl = segment_label.reshape(-1).astype(jnp.int32)
    out = _emb_lookup(idx, lbl, emb_table, seg_table)
    return out[:, :D].reshape(B, L, D)


# speedup vs baseline: 7.6451x; 7.6451x over previous
import functools

import jax
import jax.numpy as jnp
from jax import lax
from jax.experimental import pallas as pl
from jax.experimental.pallas import tpu as pltpu
from jax.experimental.pallas import tpu_sc as plsc

VOCAB = 1000000
D = 64
DP = 128
B = 4096
L = 200
N = B * L

NC = 2
NS = 16
NW = NC * NS
PER_W = N // NW
NBUF = 2
CHUNK = 256
GRP = CHUNK // 16
N_CHUNKS = PER_W // CHUNK
N_ITERS = N_CHUNKS // NBUF


def _emb_body(idx_hbm, lbl_hbm, emb_hbm, seg_hbm, out_hbm,
              idx_v, lbl_v, tok_v, seg_sm, sem_g, sem_w):
    wid = lax.axis_index("s") * NC + lax.axis_index("c")
    base = wid * PER_W

    pltpu.sync_copy(idx_hbm.at[pl.ds(base, PER_W)], idx_v)
    pltpu.sync_copy(lbl_hbm.at[pl.ds(base, PER_W)], lbl_v)
    pltpu.sync_copy(seg_hbm, seg_sm)

    def fire(g, b):
        off = g * CHUNK

        def fire_one(p, c):
            sl = pl.ds(off + p * 16, 16)
            dl = pl.ds(p * 16, 16)
            pltpu.async_copy(emb_hbm.at[idx_v[sl]], tok_v.at[b, dl],
                             sem_g.at[b])
            return c

        lax.fori_loop(0, GRP, fire_one, 0)

    lanes = lax.broadcasted_iota(jnp.int32, (16,), 0)

    def finish(g, b):
        pltpu.make_async_copy(emb_hbm.at[idx_v.at[pl.ds(0, CHUNK)]],
                              tok_v.at[b], sem_g.at[b]).wait()

        off = g * CHUNK

        def add_step(p, c):
            lblb = plsc.load_gather(lbl_v, [jnp.broadcast_to(off + p, (16,))])
            for k in range(D // 16):
                sl = pl.ds(k * 16, 16)
                sval = plsc.load_gather(seg_sm, [lblb, lanes + (k * 16)])
                tok_v[b, p, sl] = tok_v[b, p, sl] + sval
            return c

        lax.fori_loop(0, CHUNK, add_step, 0, unroll=2)
        start = base + g * CHUNK
        pltpu.async_copy(tok_v.at[b], out_hbm.at[pl.ds(start, CHUNK)],
                         sem_w.at[b]).wait()

    fire(0, 0)

    def step(i, carry):
        g0 = i * NBUF
        fire(g0 + 1, 1)
        finish(g0, 0)

        @pl.when(i + 1 < N_ITERS)
        def _():
            fire(g0 + 2, 0)

        finish(g0 + 1, 1)
        return carry

    lax.fori_loop(0, N_ITERS, step, 0)


@jax.jit
def _emb_lookup(idx, lbl, emb_table, seg_table):
    embp = jnp.pad(emb_table, ((0, 0), (0, DP - D)))
    segp = jnp.pad(seg_table, ((0, 0), (0, DP - D)))
    mesh = plsc.VectorSubcoreMesh(core_axis_name="c", subcore_axis_name="s")
    f = pl.kernel(
        _emb_body,
        out_type=jax.ShapeDtypeStruct((N, DP), jnp.float32),
        mesh=mesh,
        scratch_types=[
            pltpu.VMEM((PER_W,), jnp.int32),
            pltpu.VMEM((PER_W,), jnp.int32),
            pltpu.VMEM((NBUF, CHUNK, DP), jnp.float32),
            pltpu.VMEM((3, DP), jnp.float32),
            pltpu.SemaphoreType.DMA((NBUF,)),
            pltpu.SemaphoreType.DMA((NBUF,)),
        ],
        compiler_params=pltpu.CompilerParams(needs_layout_passes=False),
    )
    return f(idx, lbl, embp, segp)


def kernel(x, segment_label, emb_table, seg_table):
    idx = x.reshape(-1).astype(jnp.int32)
    lbl = segment_label.reshape(-1).astype(jnp.int32)
    out = _emb_lookup(idx, lbl, emb_table, seg_table)
    return out[:, :D].reshape(B, L, D)


# speedup vs baseline: 7.6583x; 1.0017x over previous
import functools

import jax
import jax.numpy as jnp
from jax import lax
from jax.experimental import pallas as pl
from jax.experimental.pallas import tpu as pltpu
from jax.experimental.pallas import tpu_sc as plsc

VOCAB = 1000000
D = 64
DP = 128
B = 4096
L = 200
N = B * L

NC = 2
NS = 16
NW = NC * NS
PER_W = N // NW
NBUF = 2
CHUNK = 256
GRP = CHUNK // 16
N_CHUNKS = PER_W // CHUNK
N_ITERS = N_CHUNKS // NBUF


def _emb_body(idx_hbm, lbl_hbm, emb_hbm, seg_hbm, out_hbm,
              idx_v, lbl_v, tok_v, seg_sm, sem_g, sem_w):
    wid = lax.axis_index("s") * NC + lax.axis_index("c")
    base = wid * PER_W

    pltpu.sync_copy(idx_hbm.at[pl.ds(base, PER_W)], idx_v)
    pltpu.sync_copy(lbl_hbm.at[pl.ds(base, PER_W)], lbl_v)
    pltpu.sync_copy(seg_hbm, seg_sm)

    def fire(g, b):
        @pl.when(g >= NBUF)
        def _():
            pltpu.make_async_copy(tok_v.at[b], out_hbm.at[pl.ds(0, CHUNK)],
                                  sem_w.at[b]).wait()

        off = g * CHUNK

        def fire_one(p, c):
            sl = pl.ds(off + p * 16, 16)
            dl = pl.ds(p * 16, 16)
            pltpu.async_copy(emb_hbm.at[idx_v[sl]], tok_v.at[b, dl],
                             sem_g.at[b])
            return c

        lax.fori_loop(0, GRP, fire_one, 0)

    lanes = lax.broadcasted_iota(jnp.int32, (16,), 0)

    def finish(g, b):
        pltpu.make_async_copy(emb_hbm.at[idx_v.at[pl.ds(0, CHUNK)]],
                              tok_v.at[b], sem_g.at[b]).wait()

        off = g * CHUNK

        def add_step(p, c):
            lblb = plsc.load_gather(lbl_v, [jnp.broadcast_to(off + p, (16,))])
            for k in range(D // 16):
                sl = pl.ds(k * 16, 16)
                sval = plsc.load_gather(seg_sm, [lblb, lanes + (k * 16)])
                tok_v[b, p, sl] = tok_v[b, p, sl] + sval
            return c

        lax.fori_loop(0, CHUNK, add_step, 0, unroll=2)
        start = base + g * CHUNK
        pltpu.async_copy(tok_v.at[b], out_hbm.at[pl.ds(start, CHUNK)],
                         sem_w.at[b])

    fire(0, 0)

    def step(i, carry):
        g0 = i * NBUF
        fire(g0 + 1, 1)
        finish(g0, 0)

        @pl.when(i + 1 < N_ITERS)
        def _():
            fire(g0 + 2, 0)

        finish(g0 + 1, 1)
        return carry

    lax.fori_loop(0, N_ITERS, step, 0)

    for b in range(NBUF):
        pltpu.make_async_copy(tok_v.at[b], out_hbm.at[pl.ds(0, CHUNK)],
                              sem_w.at[b]).wait()


@jax.jit
def _emb_lookup(idx, lbl, emb_table, seg_table):
    embp = jnp.pad(emb_table, ((0, 0), (0, DP - D)))
    segp = jnp.pad(seg_table, ((0, 0), (0, DP - D)))
    mesh = plsc.VectorSubcoreMesh(core_axis_name="c", subcore_axis_name="s")
    f = pl.kernel(
        _emb_body,
        out_type=jax.ShapeDtypeStruct((N, DP), jnp.float32),
        mesh=mesh,
        scratch_types=[
            pltpu.VMEM((PER_W,), jnp.int32),
            pltpu.VMEM((PER_W,), jnp.int32),
            pltpu.VMEM((NBUF, CHUNK, DP), jnp.float32),
            pltpu.VMEM((3, DP), jnp.float32),
            pltpu.SemaphoreType.DMA((NBUF,)),
            pltpu.SemaphoreType.DMA((NBUF,)),
        ],
        compiler_params=pltpu.CompilerParams(needs_layout_passes=False),
    )
    return f(idx, lbl, embp, segp)


def kernel(x, segment_label, emb_table, seg_table):
    idx = x.reshape(-1).astype(jnp.int32)
    lbl = segment_label.reshape(-1).astype(jnp.int32)
    out = _emb_lookup(idx, lbl, emb_table, seg_table)
    return out[:, :D].reshape(B, L, D)


# speedup vs baseline: 11.7463x; 1.5338x over previous
import functools

import jax
import jax.numpy as jnp
from jax import lax
from jax.experimental import pallas as pl
from jax.experimental.pallas import tpu as pltpu
from jax.experimental.pallas import tpu_sc as plsc

VOCAB = 1000000
D = 64
DP = 128
B = 4096
L = 200
N = B * L

NC = 2
NS = 16
NW = NC * NS
PER_W = N // NW
NBUF = 2
CHUNK = 256
GRP = CHUNK // 16
N_CHUNKS = PER_W // CHUNK
N_ITERS = N_CHUNKS // NBUF


def _emb_body(idx_hbm, lbl_hbm, emb_hbm, seg_hbm, out_hbm,
              idx_v, lbl_v, tok_v, seg_sm, sem_g, sem_w):
    wid = lax.axis_index("s") * NC + lax.axis_index("c")
    base = wid * PER_W

    pltpu.sync_copy(idx_hbm.at[pl.ds(base, PER_W)], idx_v)
    pltpu.sync_copy(lbl_hbm.at[pl.ds(base, PER_W)], lbl_v)
    pltpu.sync_copy(seg_hbm, seg_sm)

    def fire(g, b):
        @pl.when(g >= NBUF)
        def _():
            pltpu.make_async_copy(tok_v.at[b], out_hbm.at[pl.ds(0, CHUNK)],
                                  sem_w.at[b]).wait()

        off = g * CHUNK

        def fire_one(p, c):
            sl = pl.ds(off + p * 16, 16)
            dl = pl.ds(p * 16, 16)
            pltpu.async_copy(emb_hbm.at[idx_v[sl]], tok_v.at[b, dl],
                             sem_g.at[b])
            return c

        lax.fori_loop(0, GRP, fire_one, 0)

    seg_r = [[seg_sm[r, pl.ds(k * 16, 16)] for k in range(D // 16)]
             for r in range(3)]

    def finish(g, b):
        pltpu.make_async_copy(emb_hbm.at[idx_v.at[pl.ds(0, CHUNK)]],
                              tok_v.at[b], sem_g.at[b]).wait()

        off = g * CHUNK

        def add_group(grp, c):
            lbl16 = lbl_v[pl.ds(off + grp * 16, 16)]
            for j in range(16):
                p = grp * 16 + j
                lblb = lax.gather(
                    lbl16, jnp.full((16, 1), j, jnp.int32),
                    lax.GatherDimensionNumbers(
                        offset_dims=(), collapsed_slice_dims=(0,),
                        start_index_map=(0,)),
                    slice_sizes=(1,),
                    mode=lax.GatherScatterMode.PROMISE_IN_BOUNDS)
                m1 = lblb == 1
                m2 = lblb == 2
                for k in range(D // 16):
                    sl = pl.ds(k * 16, 16)
                    sval = jnp.where(m1, seg_r[1][k],
                                     jnp.where(m2, seg_r[2][k], seg_r[0][k]))
                    tok_v[b, p, sl] = tok_v[b, p, sl] + sval
            return c

        lax.fori_loop(0, GRP, add_group, 0)
        start = base + g * CHUNK
        pltpu.async_copy(tok_v.at[b], out_hbm.at[pl.ds(start, CHUNK)],
                         sem_w.at[b])

    fire(0, 0)

    def step(i, carry):
        g0 = i * NBUF
        fire(g0 + 1, 1)
        finish(g0, 0)

        @pl.when(i + 1 < N_ITERS)
        def _():
            fire(g0 + 2, 0)

        finish(g0 + 1, 1)
        return carry

    lax.fori_loop(0, N_ITERS, step, 0)

    for b in range(NBUF):
        pltpu.make_async_copy(tok_v.at[b], out_hbm.at[pl.ds(0, CHUNK)],
                              sem_w.at[b]).wait()


@jax.jit
def _emb_lookup(idx, lbl, emb_table, seg_table):
    embp = jnp.pad(emb_table, ((0, 0), (0, DP - D)))
    segp = jnp.pad(seg_table, ((0, 0), (0, DP - D)))
    mesh = plsc.VectorSubcoreMesh(core_axis_name="c", subcore_axis_name="s")
    f = pl.kernel(
        _emb_body,
        out_type=jax.ShapeDtypeStruct((N, DP), jnp.float32),
        mesh=mesh,
        scratch_types=[
            pltpu.VMEM((PER_W,), jnp.int32),
            pltpu.VMEM((PER_W,), jnp.int32),
            pltpu.VMEM((NBUF, CHUNK, DP), jnp.float32),
            pltpu.VMEM((3, DP), jnp.float32),
            pltpu.SemaphoreType.DMA((NBUF,)),
            pltpu.SemaphoreType.DMA((NBUF,)),
        ],
        compiler_params=pltpu.CompilerParams(needs_layout_passes=False),
    )
    return f(idx, lbl, embp, segp)


def kernel(x, segment_label, emb_table, seg_table):
    idx = x.reshape(-1).astype(jnp.int32)
    lbl = segment_label.reshape(-1).astype(jnp.int32)
    out = _emb_lookup(idx, lbl, emb_table, seg_table)
    return out[:, :D].reshape(B, L, D)


# speedup vs baseline: 11.7503x; 1.0003x over previous
import jax
import jax.numpy as jnp
from jax import lax
from jax.experimental import pallas as pl
from jax.experimental.pallas import tpu as pltpu
from jax.experimental.pallas import tpu_sc as plsc

VOCAB = 1000000
D = 64
DP = 128
B = 4096
L = 200
N = B * L

NC = 2
NS = 16
NW = NC * NS
PER_W = N // NW
NBUF = 2
CHUNK = 256
GRP = CHUNK // 16
N_CHUNKS = PER_W // CHUNK
N_ITERS = N_CHUNKS // NBUF


def _emb_body(idx_hbm, lbl_hbm, emb_hbm, seg_hbm, out_hbm,
              idx_v, lbl_v, tok_v, seg_sm, sem_g, sem_w):
    wid = lax.axis_index("s") * NC + lax.axis_index("c")
    base = wid * PER_W

    pltpu.sync_copy(idx_hbm.at[pl.ds(base, PER_W)], idx_v)
    pltpu.sync_copy(lbl_hbm.at[pl.ds(base, PER_W)], lbl_v)
    pltpu.sync_copy(seg_hbm, seg_sm)

    def fire(g, b):
        @pl.when(g >= NBUF)
        def _():
            pltpu.make_async_copy(tok_v.at[b], out_hbm.at[pl.ds(0, CHUNK)],
                                  sem_w.at[b]).wait()

        off = g * CHUNK

        def fire_one(p, c):
            sl = pl.ds(off + p * 16, 16)
            dl = pl.ds(p * 16, 16)
            pltpu.async_copy(emb_hbm.at[idx_v[sl]], tok_v.at[b, dl],
                             sem_g.at[b])
            return c

        lax.fori_loop(0, GRP, fire_one, 0)

    seg_r = [[seg_sm[r, pl.ds(k * 16, 16)] for k in range(D // 16)]
             for r in range(3)]

    def finish(g, b):
        pltpu.make_async_copy(emb_hbm.at[idx_v.at[pl.ds(0, CHUNK)]],
                              tok_v.at[b], sem_g.at[b]).wait()

        off = g * CHUNK

        def add_group(grp, c):
            lbl16 = lbl_v[pl.ds(off + grp * 16, 16)]
            for j in range(16):
                p = grp * 16 + j
                lblb = lax.gather(
                    lbl16, jnp.full((16, 1), j, jnp.int32),
                    lax.GatherDimensionNumbers(
                        offset_dims=(), collapsed_slice_dims=(0,),
                        start_index_map=(0,)),
                    slice_sizes=(1,),
                    mode=lax.GatherScatterMode.PROMISE_IN_BOUNDS)
                m1 = lblb == 1
                m2 = lblb == 2
                for k in range(D // 16):
                    sl = pl.ds(k * 16, 16)
                    sval = jnp.where(m1, seg_r[1][k],
                                     jnp.where(m2, seg_r[2][k], seg_r[0][k]))
                    tok_v[b, p, sl] = tok_v[b, p, sl] + sval
            return c

        lax.fori_loop(0, GRP, add_group, 0)
        start = base + g * CHUNK
        pltpu.async_copy(tok_v.at[b], out_hbm.at[pl.ds(start, CHUNK)],
                         sem_w.at[b])

    fire(0, 0)

    def step(i, carry):
        g0 = i * NBUF
        fire(g0 + 1, 1)
        finish(g0, 0)

        @pl.when(i + 1 < N_ITERS)
        def _():
            fire(g0 + 2, 0)

        finish(g0 + 1, 1)
        return carry

    lax.fori_loop(0, N_ITERS, step, 0)

    for b in range(NBUF):
        pltpu.make_async_copy(tok_v.at[b], out_hbm.at[pl.ds(0, CHUNK)],
                              sem_w.at[b]).wait()


@jax.jit
def _emb_lookup(idx, lbl, emb_table, seg_table):
    embp = jnp.pad(emb_table, ((0, 0), (0, DP - D)))
    segp = jnp.pad(seg_table, ((0, 0), (0, DP - D)))
    mesh = plsc.VectorSubcoreMesh(core_axis_name="c", subcore_axis_name="s")
    f = pl.kernel(
        _emb_body,
        out_type=jax.ShapeDtypeStruct((N, DP), jnp.float32),
        mesh=mesh,
        scratch_types=[
            pltpu.VMEM((PER_W,), jnp.int32),
            pltpu.VMEM((PER_W,), jnp.int32),
            pltpu.VMEM((NBUF, CHUNK, DP), jnp.float32),
            pltpu.VMEM((3, DP), jnp.float32),
            pltpu.SemaphoreType.DMA((NBUF,)),
            pltpu.SemaphoreType.DMA((NBUF,)),
        ],
        compiler_params=pltpu.CompilerParams(needs_layout_passes=False),
    )
    return f(idx, lbl, embp, segp)


def kernel(x, segment_label, emb_table, seg_table):
    idx = x.reshape(-1).astype(jnp.int32)
    lbl = segment_label.reshape(-1).astype(jnp.int32)
    out = _emb_lookup(idx, lbl, emb_table, seg_table)
    return out[:, :D].reshape(B, L, D)
